# Initial kernel scaffold; baseline (speedup 1.0000x reference)
import jax
import jax.numpy as jnp
from jax import lax
from jax.experimental import pallas as pl
from jax.experimental.pallas import tpu as pltpu
from jax.experimental.pallas import tpu_sc as plsc

VOCAB = 1000000
EMBED_DIM = 32
BATCH = 4096
HIST = 200

NUM_CORES = 2
NUM_SUBCORES = 16
NUM_WORKERS = NUM_CORES * NUM_SUBCORES

TOTAL = BATCH * HIST
PER_WORKER = TOTAL // NUM_WORKERS
CHUNK = 1024
CHUNKS_PER_WORKER = PER_WORKER // CHUNK


def _gather_body(table_hbm, idx_hbm, out_hbm, idx_v, rows_v, sem):
    wid = lax.axis_index("s") * NUM_CORES + lax.axis_index("c")
    base = wid * PER_WORKER

    def step(i, carry):
        off = base + i * CHUNK
        pltpu.sync_copy(idx_hbm.at[pl.ds(off, CHUNK)], idx_v)
        pltpu.async_copy(table_hbm.at[idx_v], rows_v, sem).wait()
        pltpu.sync_copy(rows_v, out_hbm.at[pl.ds(off, CHUNK)])
        return carry

    lax.fori_loop(0, CHUNKS_PER_WORKER, step, 0)


_mesh = plsc.VectorSubcoreMesh(core_axis_name="c", subcore_axis_name="s")

_gather = pl.kernel(
    _gather_body,
    out_type=jax.ShapeDtypeStruct((TOTAL, EMBED_DIM), jnp.float32),
    mesh=_mesh,
    scratch_types=[
        pltpu.VMEM((CHUNK,), jnp.int32),
        pltpu.VMEM((CHUNK, EMBED_DIM), jnp.float32),
        pltpu.SemaphoreType.DMA,
    ],
)


@jax.jit
def kernel(indices, table):
    flat = indices.reshape(TOTAL)
    out = _gather(table, flat)
    return out.reshape(BATCH, HIST, EMBED_DIM)


# speedup vs baseline: 1.4579x; 1.4579x over previous
import jax
import jax.numpy as jnp
from jax import lax
from jax.experimental import pallas as pl
from jax.experimental.pallas import tpu as pltpu
from jax.experimental.pallas import tpu_sc as plsc

VOCAB = 1000000
EMBED_DIM = 32
BATCH = 4096
HIST = 200

NUM_CORES = 2
NUM_SUBCORES = 16
NUM_WORKERS = NUM_CORES * NUM_SUBCORES

TOTAL = BATCH * HIST
PER_WORKER = TOTAL // NUM_WORKERS
CHUNK = 1024
CHUNKS_PER_WORKER = PER_WORKER // CHUNK


def _gather_body(table_hbm, idx_hbm, out_hbm, idx_v, rows_v, sem):
    wid = lax.axis_index("s") * NUM_CORES + lax.axis_index("c")
    base = wid * PER_WORKER

    def step(i, carry):
        off = base + i * CHUNK
        pltpu.sync_copy(idx_hbm.at[pl.ds(off, CHUNK)], idx_v)
        pltpu.async_copy(table_hbm.at[idx_v], rows_v, sem).wait()
        pltpu.sync_copy(rows_v, out_hbm.at[pl.ds(off, CHUNK)])
        return carry

    lax.fori_loop(0, CHUNKS_PER_WORKER, step, 0)


_mesh = plsc.VectorSubcoreMesh(core_axis_name="c", subcore_axis_name="s")

_gather = pl.kernel(
    _gather_body,
    out_type=jax.ShapeDtypeStruct((TOTAL, EMBED_DIM), jnp.float32),
    mesh=_mesh,
    scratch_types=[
        pltpu.VMEM((CHUNK,), jnp.int32),
        pltpu.VMEM((CHUNK, EMBED_DIM), jnp.float32),
        pltpu.SemaphoreType.DMA,
    ],
    compiler_params=pltpu.CompilerParams(use_tc_tiling_on_sc=False),
)


@jax.jit
def kernel(indices, table):
    flat = indices.reshape(TOTAL)
    out = _gather(table, flat)
    return out.reshape(BATCH, HIST, EMBED_DIM)


# speedup vs baseline: 1.4970x; 1.0268x over previous
import jax
import jax.numpy as jnp
from jax import lax
from jax.experimental import pallas as pl
from jax.experimental.pallas import tpu as pltpu
from jax.experimental.pallas import tpu_sc as plsc

VOCAB = 1000000
EMBED_DIM = 32
BATCH = 4096
HIST = 200

NUM_CORES = 2
NUM_SUBCORES = 16
NUM_WORKERS = NUM_CORES * NUM_SUBCORES

TOTAL = BATCH * HIST
PER_WORKER = TOTAL // NUM_WORKERS
CHUNK = 1280
NCHUNKS = PER_WORKER // CHUNK
NBUF = 2
OUTER = NCHUNKS // NBUF


def _gather_body(table_hbm, idx_hbm, out_hbm, *scratch):
    idx_v = scratch[0:NBUF]
    rows_v = scratch[NBUF:2 * NBUF]
    idx_sems = scratch[2 * NBUF:3 * NBUF]
    gat_sems = scratch[3 * NBUF:4 * NBUF]
    st_sems = scratch[4 * NBUF:5 * NBUF]

    wid = lax.axis_index("s") * NUM_CORES + lax.axis_index("c")
    base = wid * PER_WORKER

    def idx_start(g, b):
        pltpu.make_async_copy(
            idx_hbm.at[pl.ds(base + g * CHUNK, CHUNK)], idx_v[b], idx_sems[b]
        ).start()

    def idx_wait(b):
        pltpu.make_async_copy(
            idx_hbm.at[pl.ds(base, CHUNK)], idx_v[b], idx_sems[b]
        ).wait()

    def gather_start(b):
        pltpu.make_async_copy(table_hbm.at[idx_v[b]], rows_v[b], gat_sems[b]).start()

    def gather_wait(b):
        pltpu.make_async_copy(table_hbm.at[idx_v[b]], rows_v[b], gat_sems[b]).wait()

    def store_start(g, b):
        pltpu.make_async_copy(
            rows_v[b], out_hbm.at[pl.ds(base + g * CHUNK, CHUNK)], st_sems[b]
        ).start()

    def store_wait(b):
        pltpu.make_async_copy(
            rows_v[b], out_hbm.at[pl.ds(base, CHUNK)], st_sems[b]
        ).wait()

    for b in range(NBUF):
        idx_start(b, b)

    def outer_step(o, carry):
        for b in range(NBUF):
            g = o * NBUF + b
            def retire(c, cb):
                gather_wait(cb)
                store_start(c, cb)

                @pl.when(c + NBUF < NCHUNKS)
                def _():
                    idx_start(c + NBUF, cb)

            if b == 0:
                @pl.when(o > 0)
                def _():
                    retire(g - 1, NBUF - 1)
            else:
                retire(g - 1, b - 1)

            @pl.when(o > 0)
            def _():
                store_wait(b)

            idx_wait(b)
            gather_start(b)
        return carry

    lax.fori_loop(0, OUTER, outer_step, 0)

    last_b = (NCHUNKS - 1) % NBUF
    gather_wait(last_b)
    store_start(NCHUNKS - 1, last_b)
    for b in range(NBUF):
        store_wait(b)


_mesh = plsc.VectorSubcoreMesh(core_axis_name="c", subcore_axis_name="s")

_scratch = (
    [pltpu.VMEM((CHUNK,), jnp.int32) for _ in range(NBUF)]
    + [pltpu.VMEM((CHUNK, EMBED_DIM), jnp.float32) for _ in range(NBUF)]
    + [pltpu.SemaphoreType.DMA for _ in range(3 * NBUF)]
)

_gather = pl.kernel(
    _gather_body,
    out_type=jax.ShapeDtypeStruct((TOTAL, EMBED_DIM), jnp.float32),
    mesh=_mesh,
    scratch_types=_scratch,
    compiler_params=pltpu.CompilerParams(use_tc_tiling_on_sc=False),
)


@jax.jit
def kernel(indices, table):
    flat = indices.reshape(TOTAL)
    out = _gather(table, flat)
    return out.reshape(BATCH, HIST, EMBED_DIM)


# speedup vs baseline: 1.4994x; 1.0016x over previous
import jax
import jax.numpy as jnp
from jax import lax
from jax.experimental import pallas as pl
from jax.experimental.pallas import tpu as pltpu
from jax.experimental.pallas import tpu_sc as plsc

VOCAB = 1000000
EMBED_DIM = 32
BATCH = 4096
HIST = 200

NUM_CORES = 2
NUM_SUBCORES = 16
NUM_WORKERS = NUM_CORES * NUM_SUBCORES

TOTAL = BATCH * HIST
PER_WORKER = TOTAL // NUM_WORKERS
CHUNK = 1280
NCHUNKS = PER_WORKER // CHUNK
NBUF = 2
OUTER = NCHUNKS // NBUF
NSTREAMS = 4
SUB = CHUNK // NSTREAMS


def _gather_body(table_hbm, idx_hbm, out_hbm, *scratch):
    idx_v = scratch[0:NBUF]
    rows_v = scratch[NBUF:2 * NBUF]
    idx_sems = scratch[2 * NBUF:3 * NBUF]
    gat_sems = scratch[3 * NBUF:3 * NBUF + NBUF * NSTREAMS]
    st_sems = scratch[3 * NBUF + NBUF * NSTREAMS:]

    wid = lax.axis_index("s") * NUM_CORES + lax.axis_index("c")
    base = wid * PER_WORKER

    def idx_start(g, b):
        pltpu.make_async_copy(
            idx_hbm.at[pl.ds(base + g * CHUNK, CHUNK)], idx_v[b], idx_sems[b]
        ).start()

    def idx_wait(b):
        pltpu.make_async_copy(
            idx_hbm.at[pl.ds(base, CHUNK)], idx_v[b], idx_sems[b]
        ).wait()

    def gather_start(b):
        for s in range(NSTREAMS):
            pltpu.make_async_copy(
                table_hbm.at[idx_v[b].at[pl.ds(s * SUB, SUB)]],
                rows_v[b].at[pl.ds(s * SUB, SUB)],
                gat_sems[b * NSTREAMS + s],
            ).start()

    def gather_wait(b):
        for s in range(NSTREAMS):
            pltpu.make_async_copy(
                table_hbm.at[idx_v[b].at[pl.ds(s * SUB, SUB)]],
                rows_v[b].at[pl.ds(s * SUB, SUB)],
                gat_sems[b * NSTREAMS + s],
            ).wait()

    def store_start(g, b):
        pltpu.make_async_copy(
            rows_v[b], out_hbm.at[pl.ds(base + g * CHUNK, CHUNK)], st_sems[b]
        ).start()

    def store_wait(b):
        pltpu.make_async_copy(
            rows_v[b], out_hbm.at[pl.ds(base, CHUNK)], st_sems[b]
        ).wait()

    for b in range(NBUF):
        idx_start(b, b)

    def outer_step(o, carry):
        for b in range(NBUF):
            g = o * NBUF + b
            def retire(c, cb):
                gather_wait(cb)
                store_start(c, cb)

                @pl.when(c + NBUF < NCHUNKS)
                def _():
                    idx_start(c + NBUF, cb)

            if b == 0:
                @pl.when(o > 0)
                def _():
                    retire(g - 1, NBUF - 1)
            else:
                retire(g - 1, b - 1)

            @pl.when(o > 0)
            def _():
                store_wait(b)

            idx_wait(b)
            gather_start(b)
        return carry

    lax.fori_loop(0, OUTER, outer_step, 0)

    last_b = (NCHUNKS - 1) % NBUF
    gather_wait(last_b)
    store_start(NCHUNKS - 1, last_b)
    for b in range(NBUF):
        store_wait(b)


_mesh = plsc.VectorSubcoreMesh(core_axis_name="c", subcore_axis_name="s")

_scratch = (
    [pltpu.VMEM((CHUNK,), jnp.int32) for _ in range(NBUF)]
    + [pltpu.VMEM((CHUNK, EMBED_DIM), jnp.float32) for _ in range(NBUF)]
    + [pltpu.SemaphoreType.DMA for _ in range(2 * NBUF + NBUF * NSTREAMS)]
)

_gather = pl.kernel(
    _gather_body,
    out_type=jax.ShapeDtypeStruct((TOTAL, EMBED_DIM), jnp.float32),
    mesh=_mesh,
    scratch_types=_scratch,
    compiler_params=pltpu.CompilerParams(use_tc_tiling_on_sc=False),
)


@jax.jit
def kernel(indices, table):
    flat = indices.reshape(TOTAL)
    out = _gather(table, flat)
    return out.reshape(BATCH, HIST, EMBED_DIM)
